# baseline (speedup 1.0000x reference)
import functools

import jax
import jax.numpy as jnp
from jax import lax
from jax.experimental import pallas as pl
from jax.experimental.pallas import tpu as pltpu
from jax.experimental.pallas import tpu_sc as plsc

f32 = jnp.float32
i32 = jnp.int32

VOCAB = 65
T = 8
ROWB = 72
NKEY = ROWB * T
DPAD = 80
LSECOL = 65
NTOK = 1024
NC = 2
NS = 16
NW = NC * NS
BPT = NTOK // NW


def _tc_body(tok_ref, pos_ref, w_ref, b_ref, ctab_ref):
    W = w_ref[...]
    Lt = jnp.dot(tok_ref[...], W, preferred_element_type=f32) + b_ref[...]
    Lp = jnp.dot(pos_ref[...], W, preferred_element_type=f32)
    col = lax.broadcasted_iota(i32, (ROWB, DPAD), 1)
    neg = jnp.float32(-1e30)
    for t in range(T):
        blk = Lt + Lp[t:t + 1, :]
        xm = jnp.where(col < VOCAB, blk, neg)
        m = jnp.max(xm, axis=1, keepdims=True)
        s = jnp.sum(jnp.exp(xm - m), axis=1, keepdims=True)
        lse = m + jnp.log(s)
        ctab_ref[pl.ds(t * ROWB, ROWB), :] = jnp.where(col < VOCAB, blk, lse)


_tc_tables = pl.pallas_call(
    _tc_body,
    out_shape=jax.ShapeDtypeStruct((NKEY, DPAD), f32),
)


@functools.partial(
    pl.kernel,
    mesh=plsc.VectorSubcoreMesh(core_axis_name="c", subcore_axis_name="s"),
    out_type=(
        jax.ShapeDtypeStruct((NTOK, DPAD), f32),
        jax.ShapeDtypeStruct((NC, 16), f32),
    ),
    scratch_types=[
        pltpu.VMEM((BPT,), i32),
        pltpu.VMEM((BPT,), i32),
        pltpu.VMEM((BPT,), i32),
        pltpu.VMEM((BPT, DPAD), f32),
        pltpu.VMEM((16,), f32),
        pltpu.VMEM((NS, 16), f32),
        pltpu.VMEM((16,), f32),
        pltpu.VMEM_SHARED((NS, 16), f32),
        pltpu.SemaphoreType.DMA,
    ],
    compiler_params=pltpu.CompilerParams(
        needs_layout_passes=False, use_tc_tiling_on_sc=False),
)
def _sc_kernel(ctab, idxf, tgtf, out, loss_part,
               idx_v, tgt_v, keys_v, rows_v, acc_ref, sums_v, loss_v,
               shared, sem):
    cid = lax.axis_index("c")
    sid = lax.axis_index("s")
    wid = sid * NC + cid
    base = wid * BPT

    pltpu.sync_copy(idxf.at[pl.ds(base, BPT)], idx_v)
    pltpu.sync_copy(tgtf.at[pl.ds(base, BPT)], tgt_v)

    lane = jnp.arange(16, dtype=i32)
    pos = lax.bitwise_and(lane, T - 1)
    for c in range(BPT // 16):
        keys_v[pl.ds(c * 16, 16)] = pos * ROWB + idx_v[pl.ds(c * 16, 16)]

    pltpu.async_copy(ctab.at[keys_v], rows_v, sem).wait()
    pltpu.sync_copy(rows_v, out.at[pl.ds(base, BPT)])

    lsecol = jnp.full((16,), LSECOL, dtype=i32)
    acc = jnp.zeros((16,), f32)
    for c in range(BPT // 16):
        rowi = lane + c * 16
        lg = plsc.load_gather(rows_v, [rowi, lsecol])
        picked = plsc.load_gather(
            rows_v, [rowi, tgt_v[pl.ds(c * 16, 16)]])
        acc = acc + (lg - picked)
    acc_ref[...] = acc

    pltpu.sync_copy(acc_ref, shared.at[sid])
    plsc.subcore_barrier()

    @pl.when(sid == 0)
    def _():
        pltpu.sync_copy(shared, sums_v)
        tot = jnp.zeros((16,), f32)
        for i in range(NS):
            tot = tot + sums_v[i]
        total = jnp.sum(tot) * jnp.float32(1.0 / NTOK)
        loss_v[...] = jnp.broadcast_to(total, (16,))
        pltpu.sync_copy(loss_v, loss_part.at[cid])


def kernel(idx, targets, tok_table, pos_table, W, b):
    V, NE = tok_table.shape
    tok_p = jnp.zeros((ROWB, NE), f32).at[:V].set(tok_table)
    W_p = jnp.zeros((NE, DPAD), f32).at[:, :V].set(W)
    b_p = jnp.zeros((1, DPAD), f32).at[0, :V].set(b)

    ctab = _tc_tables(tok_p, pos_table.astype(f32), W_p, b_p)

    idxf = idx.reshape(-1).astype(i32)
    tgtf = targets.reshape(-1).astype(i32)
    out_pad, loss_part = _sc_kernel(ctab, idxf, tgtf)

    logits = out_pad[:, :V]
    loss = loss_part[0, 0] + loss_part[1, 0]
    return (logits, loss)


# speedup vs baseline: 1.2993x; 1.2993x over previous
import functools

import jax
import jax.numpy as jnp
from jax import lax
from jax.experimental import pallas as pl
from jax.experimental.pallas import tpu as pltpu
from jax.experimental.pallas import tpu_sc as plsc

f32 = jnp.float32
i32 = jnp.int32

VOCAB = 65
T = 8
ROWB = 72
NKEY = ROWB * T
DPAD = 128
LSECOL = 65
NTOK = 1024
NC = 2
NS = 16
NW = NC * NS
BPT = NTOK // NW


def _tc_body(tok_ref, pos_ref, w_ref, b_ref, ctab_ref):
    W = w_ref[...]
    b = b_ref[...][None, :]
    Lt = jnp.dot(tok_ref[...], W, preferred_element_type=f32,
                 precision=lax.Precision.HIGHEST) + b
    Lp = jnp.dot(pos_ref[...], W, preferred_element_type=f32,
                 precision=lax.Precision.HIGHEST)
    for t in range(T):
        blk = Lt + Lp[t:t + 1, :]
        m = jnp.max(blk, axis=1, keepdims=True)
        s = jnp.sum(jnp.exp(blk - m), axis=1, keepdims=True)
        ctab_ref[pl.ds(t * ROWB, VOCAB), :VOCAB] = blk
        ctab_ref[pl.ds(t * ROWB, VOCAB), LSECOL:LSECOL + 1] = m + jnp.log(s)


_tc_tables = pl.pallas_call(
    _tc_body,
    out_shape=jax.ShapeDtypeStruct((NKEY, DPAD), f32),
)


@functools.partial(
    pl.kernel,
    mesh=plsc.VectorSubcoreMesh(core_axis_name="c", subcore_axis_name="s"),
    out_type=(
        jax.ShapeDtypeStruct((NTOK, DPAD), f32),
        jax.ShapeDtypeStruct((NW, 16), f32),
    ),
    scratch_types=[
        pltpu.VMEM((BPT,), i32),
        pltpu.VMEM((BPT,), i32),
        pltpu.VMEM((BPT,), i32),
        pltpu.VMEM((BPT, DPAD), f32),
        pltpu.VMEM((16,), f32),
        pltpu.SemaphoreType.DMA,
        pltpu.SemaphoreType.DMA,
        pltpu.SemaphoreType.DMA,
    ],
    compiler_params=pltpu.CompilerParams(
        needs_layout_passes=False, use_tc_tiling_on_sc=False),
)
def _sc_kernel(ctab, idxf, tgtf, out, loss_part,
               idx_v, tgt_v, keys_v, rows_v, acc_ref, sem, sem2, sem3):
    cid = lax.axis_index("c")
    sid = lax.axis_index("s")
    wid = sid * NC + cid
    base = wid * BPT

    cp_idx = pltpu.async_copy(idxf.at[pl.ds(base, BPT)], idx_v, sem)
    cp_tgt = pltpu.async_copy(tgtf.at[pl.ds(base, BPT)], tgt_v, sem2)
    cp_idx.wait()

    lane = jnp.arange(16, dtype=i32)
    pos = lax.bitwise_and(lane, T - 1)
    for c in range(BPT // 16):
        keys_v[pl.ds(c * 16, 16)] = pos * ROWB + idx_v[pl.ds(c * 16, 16)]

    pltpu.async_copy(ctab.at[keys_v], rows_v, sem3).wait()
    cp_out = pltpu.async_copy(rows_v, out.at[pl.ds(base, BPT)], sem3)

    cp_tgt.wait()
    lsecol = jnp.full((16,), LSECOL, dtype=i32)
    acc = jnp.zeros((16,), f32)
    for c in range(BPT // 16):
        rowi = lane + c * 16
        lg = plsc.load_gather(rows_v, [rowi, lsecol])
        picked = plsc.load_gather(
            rows_v, [rowi, tgt_v[pl.ds(c * 16, 16)]])
        acc = acc + (lg - picked)
    total = jnp.sum(acc) * jnp.float32(1.0 / NTOK)
    acc_ref[...] = jnp.broadcast_to(total, (16,))
    pltpu.sync_copy(acc_ref, loss_part.at[wid])
    cp_out.wait()


def kernel(idx, targets, tok_table, pos_table, W, b):
    V = tok_table.shape[0]
    ctab = _tc_tables(tok_table.astype(f32), pos_table.astype(f32),
                      W.astype(f32), b.astype(f32))

    idxf = idx.reshape(-1).astype(i32)
    tgtf = targets.reshape(-1).astype(i32)
    out_pad, loss_part = _sc_kernel(ctab, idxf, tgtf)

    logits = out_pad[:, :V]
    loss = jnp.sum(loss_part[:, 0])
    return (logits, loss)
